# Initial kernel scaffold; baseline (speedup 1.0000x reference)
import jax
import jax.numpy as jnp
from jax.experimental import pallas as pl


def kernel(a_ids, event_ids, b_ids, c_ids, ent_emb, rel_emb, proj_ab_W, proj_ab_b, proj_bc_W, proj_bc_b, ab_msg_W1, ab_msg_b1, ab_msg_W2, ab_msg_b2, ab_att_W1, ab_att_b1, ab_att_W2, ab_att_b2, ab_gru_Wih, ab_gru_Whh, ab_gru_bih, ab_gru_bhh, bc_msg_W1, bc_msg_b1, bc_msg_W2, bc_msg_b2, bc_att_W1, bc_att_b1, bc_att_W2, bc_att_b2, bc_gru_Wih, bc_gru_Whh, bc_gru_bih, bc_gru_bhh, head_ab_W1, head_ab_b1, head_ab_W2, head_ab_b2, head_bc_W1, head_bc_b1, head_bc_W2, head_bc_b2):
    raise NotImplementedError("write your pallas kernel here")


# speedup vs baseline: 1.2175x; 1.2175x over previous
import functools

import jax
import jax.numpy as jnp
from jax import lax
from jax.experimental import pallas as pl
from jax.experimental.pallas import tpu as pltpu
from jax.experimental.pallas import tpu_sc as plsc

B = 16384
EMB = 64
HID = 64
NREL = 16

_NC, _NS = 2, 16
_NW = _NC * _NS
_IDS = 4 * B
_PER_W = _IDS // _NW
_CH = 512
_NCH = _PER_W // _CH

@functools.cache
def _make_sc_gather():
    mesh = plsc.VectorSubcoreMesh(core_axis_name="c", subcore_axis_name="s")

    @functools.partial(
        pl.kernel,
        out_type=jax.ShapeDtypeStruct((_IDS, EMB), jnp.float32),
        mesh=mesh,
        scratch_types=[
            pltpu.VMEM((_PER_W,), jnp.int32),
            pltpu.VMEM((2, _CH, EMB), jnp.float32),
            pltpu.SemaphoreType.DMA,
            pltpu.SemaphoreType.DMA,
        ],
        compiler_params=pltpu.CompilerParams(use_tc_tiling_on_sc=False),
    )
    def _sc_gather(ids_hbm, table_hbm, out_hbm, idx_v, rows_v, sem0, sem1):
        wid = lax.axis_index("s") * _NC + lax.axis_index("c")
        base = wid * _PER_W
        pltpu.sync_copy(ids_hbm.at[pl.ds(base, _PER_W)], idx_v)
        sems = [sem0, sem1]
        copies = [None, None]
        copies[0] = pltpu.async_copy(
            table_hbm.at[idx_v.at[pl.ds(0, _CH)]], rows_v.at[0], sems[0])
        for c in range(_NCH):
            cur = c % 2
            if c + 1 < _NCH:
                nxt = (c + 1) % 2
                copies[nxt] = pltpu.async_copy(
                    table_hbm.at[idx_v.at[pl.ds((c + 1) * _CH, _CH)]],
                    rows_v.at[nxt], sems[nxt])
            copies[cur].wait()
            pltpu.sync_copy(rows_v.at[cur],
                            out_hbm.at[pl.ds(base + c * _CH, _CH)])

    return _sc_gather


_S = 512
_GRID = B // _S


def _sigmoid(x):
    return jax.nn.sigmoid(x)


def _gnn(x, D, w):
    hm = jnp.maximum(jnp.dot(D, w["w1mT"], preferred_element_type=jnp.float32, precision=lax.Precision.HIGHEST)
                     + w["b1m6"], 0.0)
    ha = jnp.maximum(jnp.dot(D, w["w1aT"], preferred_element_type=jnp.float32, precision=lax.Precision.HIGHEST)
                     + w["b1a6"], 0.0)
    msg = jnp.dot(hm, w["w2mT"], preferred_element_type=jnp.float32, precision=lax.Precision.HIGHEST) + w["b2m"]
    att = _sigmoid(jnp.dot(ha, w["w2aT"], preferred_element_type=jnp.float32, precision=lax.Precision.HIGHEST)
                   + w["b2a"])
    m = msg * att
    S = _S
    mA = m[S:2 * S]
    mE = m[0:S] + m[3 * S:4 * S]
    mB = m[2 * S:3 * S] + m[5 * S:6 * S]
    mC = m[4 * S:5 * S]
    sm = jnp.concatenate([mA, mE, mB, mC], axis=0)
    gi_r = jnp.dot(sm, w["wih_r"], preferred_element_type=jnp.float32, precision=lax.Precision.HIGHEST) + w["bih_r"]
    gi_z = jnp.dot(sm, w["wih_z"], preferred_element_type=jnp.float32, precision=lax.Precision.HIGHEST) + w["bih_z"]
    gi_n = jnp.dot(sm, w["wih_n"], preferred_element_type=jnp.float32, precision=lax.Precision.HIGHEST) + w["bih_n"]
    gh_r = jnp.dot(x, w["whh_r"], preferred_element_type=jnp.float32, precision=lax.Precision.HIGHEST) + w["bhh_r"]
    gh_z = jnp.dot(x, w["whh_z"], preferred_element_type=jnp.float32, precision=lax.Precision.HIGHEST) + w["bhh_z"]
    gh_n = jnp.dot(x, w["whh_n"], preferred_element_type=jnp.float32, precision=lax.Precision.HIGHEST) + w["bhh_n"]
    r = _sigmoid(gi_r + gh_r)
    z = _sigmoid(gi_z + gh_z)
    n = jnp.tanh(gi_n + r * gh_n)
    return (1.0 - z) * n + z * x


def _diffs(x):
    S = _S
    d0 = x[0:S] - x[S:2 * S]
    d2 = x[S:2 * S] - x[2 * S:3 * S]
    d4 = x[2 * S:3 * S] - x[3 * S:4 * S]
    return jnp.concatenate([d0, -d0, d2, -d2, d4, -d4], axis=0)


def _head(h, w1T, b1, w2T, b2):
    hh = jnp.maximum(jnp.dot(h, w1T, preferred_element_type=jnp.float32, precision=lax.Precision.HIGHEST) + b1,
                     0.0)
    return jnp.dot(hh, w2T, preferred_element_type=jnp.float32, precision=lax.Precision.HIGHEST) + b2


_SEG_KEYS = ("w1mT", "b1m6", "w1aT", "b1a6", "w2mT", "b2m", "w2aT", "b2a",
             "wih_r", "wih_z", "wih_n", "whh_r", "whh_z", "whh_n",
             "bih_r", "bih_z", "bih_n", "bhh_r", "bhh_z", "bhh_n")
_TC_ARG_KEYS = (
    ["node", "rel_emb", "pabT", "pab_b", "pbc_nT", "pbc_rT", "pbc_b"]
    + ["ab_" + k for k in _SEG_KEYS] + ["bc_" + k for k in _SEG_KEYS]
    + ["hab_w1T", "hab_b1", "hab_w2T", "hab_b2",
       "hbc_w1T", "hbc_b1", "hbc_w2T", "hbc_b2"])


def _tc_body(*refs):
    w = {k: r[...] for k, r in zip(_TC_ARG_KEYS, refs[:len(_TC_ARG_KEYS)])}
    ab_out, bc_out = refs[len(_TC_ARG_KEYS):]
    S = _S
    node = w["node"].reshape(4 * S, EMB)
    ab = {k[3:]: w[k] for k in w if k.startswith("ab_")}
    bc = {k[3:]: w[k] for k in w if k.startswith("bc_")}

    x_ab = jnp.maximum(
        jnp.dot(node, w["pabT"], preferred_element_type=jnp.float32, precision=lax.Precision.HIGHEST)
        + w["pab_b"], 0.0)
    s_ab = _gnn(x_ab, _diffs(x_ab), ab)
    h_ab = s_ab[0:S] - s_ab[2 * S:3 * S]
    logits_ab = _head(h_ab, w["hab_w1T"], w["hab_b1"], w["hab_w2T"],
                      w["hab_b2"])
    ab_out[...] = logits_ab

    mx = jnp.max(logits_ab, axis=1, keepdims=True)
    iota = lax.broadcasted_iota(jnp.int32, (S, NREL), 1)
    cand = jnp.where(logits_ab >= mx, iota, NREL)
    rel = jnp.min(cand, axis=1, keepdims=True)
    oh = (iota == rel).astype(jnp.float32)
    r_vec = jnp.dot(oh, w["rel_emb"], preferred_element_type=jnp.float32, precision=lax.Precision.HIGHEST)

    t = jnp.dot(r_vec, w["pbc_rT"], preferred_element_type=jnp.float32, precision=lax.Precision.HIGHEST)
    r_rep = jnp.concatenate([t, t, t, t], axis=0)
    x_bc = jnp.maximum(
        jnp.dot(node, w["pbc_nT"], preferred_element_type=jnp.float32, precision=lax.Precision.HIGHEST)
        + r_rep + w["pbc_b"], 0.0)
    s_bc = _gnn(x_bc, _diffs(x_bc), bc)
    h_bc = s_bc[2 * S:3 * S] - s_bc[3 * S:4 * S]
    bc_out[...] = _head(h_bc, w["hbc_w1T"], w["hbc_b1"], w["hbc_w2T"],
                        w["hbc_b2"])


def _tc_forward(args, interpret=False):
    def spec(k):
        a = args[k]
        if k == "node":
            return pl.BlockSpec((4, _S, EMB), lambda i: (0, i, 0))
        nd = a.ndim
        return pl.BlockSpec(a.shape, lambda i, _n=nd: (0,) * _n)

    return pl.pallas_call(
        _tc_body,
        grid=(_GRID,),
        in_specs=[spec(k) for k in _TC_ARG_KEYS],
        out_specs=[pl.BlockSpec((_S, NREL), lambda i: (i, 0)),
                   pl.BlockSpec((_S, NREL), lambda i: (i, 0))],
        out_shape=[jax.ShapeDtypeStruct((B, NREL), jnp.float32),
                   jax.ShapeDtypeStruct((B, NREL), jnp.float32)],
        interpret=interpret,
    )(*[args[k] for k in _TC_ARG_KEYS])


def _pack_seg(msg_W1, msg_b1, msg_W2, msg_b2, att_W1, att_b1, att_W2, att_b2,
              gru_Wih, gru_Whh, gru_bih, gru_bhh):
    w = {}
    w["w1mT"] = msg_W1[:, :EMB].T
    w["w1aT"] = att_W1[:, :EMB].T
    b1m = msg_b1[None, :] + msg_W1[:, EMB:EMB + 6].T
    b1a = att_b1[None, :] + att_W1[:, EMB:EMB + 6].T
    w["b1m6"] = jnp.repeat(b1m, _S, axis=0)
    w["b1a6"] = jnp.repeat(b1a, _S, axis=0)
    w["w2mT"] = msg_W2.T
    w["b2m"] = msg_b2[None, :]
    w["w2aT"] = att_W2.T
    w["b2a"] = att_b2[None, :]
    for i, g in enumerate(("r", "z", "n")):
        w["wih_" + g] = gru_Wih[i * HID:(i + 1) * HID].T
        w["whh_" + g] = gru_Whh[i * HID:(i + 1) * HID].T
        w["bih_" + g] = gru_bih[None, i * HID:(i + 1) * HID]
        w["bhh_" + g] = gru_bhh[None, i * HID:(i + 1) * HID]
    return w


def _assemble_tc_args(node4, rel_emb, p):
    args = {"node": node4, "rel_emb": rel_emb}
    args["pabT"] = p["proj_ab_W"].T
    args["pab_b"] = p["proj_ab_b"][None, :]
    args["pbc_nT"] = p["proj_bc_W"][:, :EMB].T
    args["pbc_rT"] = p["proj_bc_W"][:, EMB:].T
    args["pbc_b"] = p["proj_bc_b"][None, :]
    for pre in ("ab", "bc"):
        seg = _pack_seg(*[p[f"{pre}_{n}"] for n in (
            "msg_W1", "msg_b1", "msg_W2", "msg_b2",
            "att_W1", "att_b1", "att_W2", "att_b2",
            "gru_Wih", "gru_Whh", "gru_bih", "gru_bhh")])
        for k, v in seg.items():
            args[f"{pre}_{k}"] = v
    for pre, tag in (("head_ab", "hab"), ("head_bc", "hbc")):
        args[f"{tag}_w1T"] = p[f"{pre}_W1"].T
        args[f"{tag}_b1"] = p[f"{pre}_b1"][None, :]
        args[f"{tag}_w2T"] = p[f"{pre}_W2"].T
        args[f"{tag}_b2"] = p[f"{pre}_b2"][None, :]
    return args


def kernel(a_ids, event_ids, b_ids, c_ids, ent_emb, rel_emb,
           proj_ab_W, proj_ab_b, proj_bc_W, proj_bc_b,
           ab_msg_W1, ab_msg_b1, ab_msg_W2, ab_msg_b2,
           ab_att_W1, ab_att_b1, ab_att_W2, ab_att_b2,
           ab_gru_Wih, ab_gru_Whh, ab_gru_bih, ab_gru_bhh,
           bc_msg_W1, bc_msg_b1, bc_msg_W2, bc_msg_b2,
           bc_att_W1, bc_att_b1, bc_att_W2, bc_att_b2,
           bc_gru_Wih, bc_gru_Whh, bc_gru_bih, bc_gru_bhh,
           head_ab_W1, head_ab_b1, head_ab_W2, head_ab_b2,
           head_bc_W1, head_bc_b1, head_bc_W2, head_bc_b2):
    p = dict(locals())
    ids_all = jnp.concatenate(
        [a_ids, event_ids, b_ids, c_ids]).astype(jnp.int32)
    gathered = _make_sc_gather()(ids_all, ent_emb)
    node4 = gathered.reshape(4, B, EMB)
    args = _assemble_tc_args(node4, rel_emb, p)
    logits_ab, logits_bc = _tc_forward(args)
    return logits_ab, logits_bc


# speedup vs baseline: 1.4232x; 1.1690x over previous
import functools

import jax
import jax.numpy as jnp
from jax import lax
from jax.experimental import pallas as pl
from jax.experimental.pallas import tpu as pltpu
from jax.experimental.pallas import tpu_sc as plsc

B = 16384
EMB = 64
HID = 64
NREL = 16

_NC, _NS = 2, 16
_NW = _NC * _NS
_IDS = 4 * B
_PER_W = _IDS // _NW
_CH = 512
_NCH = _PER_W // _CH

@functools.cache
def _make_sc_gather():
    mesh = plsc.VectorSubcoreMesh(core_axis_name="c", subcore_axis_name="s")

    @functools.partial(
        pl.kernel,
        out_type=jax.ShapeDtypeStruct((_IDS, EMB), jnp.float32),
        mesh=mesh,
        scratch_types=[
            pltpu.VMEM((_PER_W,), jnp.int32),
            pltpu.VMEM((2, _CH, EMB), jnp.float32),
            pltpu.SemaphoreType.DMA,
            pltpu.SemaphoreType.DMA,
        ],
        compiler_params=pltpu.CompilerParams(use_tc_tiling_on_sc=False),
    )
    def _sc_gather(ids_hbm, table_hbm, out_hbm, idx_v, rows_v, sem0, sem1):
        wid = lax.axis_index("s") * _NC + lax.axis_index("c")
        base = wid * _PER_W
        pltpu.sync_copy(ids_hbm.at[pl.ds(base, _PER_W)], idx_v)
        sems = [sem0, sem1]
        copies = [None, None]
        copies[0] = pltpu.async_copy(
            table_hbm.at[idx_v.at[pl.ds(0, _CH)]], rows_v.at[0], sems[0])
        for c in range(_NCH):
            cur = c % 2
            if c + 1 < _NCH:
                nxt = (c + 1) % 2
                copies[nxt] = pltpu.async_copy(
                    table_hbm.at[idx_v.at[pl.ds((c + 1) * _CH, _CH)]],
                    rows_v.at[nxt], sems[nxt])
            copies[cur].wait()
            pltpu.sync_copy(rows_v.at[cur],
                            out_hbm.at[pl.ds(base + c * _CH, _CH)])

    return _sc_gather


_S = 512
_GRID = B // _S


_PREC = lax.Precision.HIGHEST


def _dot(a, b, prec=_PREC):
    return jnp.dot(a, b, preferred_element_type=jnp.float32, precision=prec)


def _sigmoid(x):
    return jax.nn.sigmoid(x)


def _gnn(x, w):
    S = _S
    d0 = x[0:S] - x[S:2 * S]
    d2 = x[S:2 * S] - x[2 * S:3 * S]
    d4 = x[2 * S:3 * S] - x[3 * S:4 * S]
    D = jnp.concatenate([d0, -d0, d2, -d2, d4, -d4], axis=0)
    H = jnp.maximum(_dot(D, w["w1cat"]) + w["b1cat6"], 0.0)
    Z = _dot(H, w["w2bd"]) + w["b2cat"]
    m = Z[:, 0:HID] * _sigmoid(Z[:, HID:2 * HID])
    mA = m[S:2 * S]
    mE = m[0:S] + m[3 * S:4 * S]
    mB = m[2 * S:3 * S] + m[5 * S:6 * S]
    mC = m[4 * S:5 * S]
    sm = jnp.concatenate([mA, mE, mB, mC], axis=0)
    gx = jnp.concatenate([sm, x], axis=1)
    G = _dot(gx, w["wg"]) + w["bg"]
    r = _sigmoid(G[:, 0:HID])
    z = _sigmoid(G[:, HID:2 * HID])
    n = jnp.tanh(G[:, 2 * HID:3 * HID] + r * G[:, 3 * HID:4 * HID])
    return (1.0 - z) * n + z * x


def _head(h, w1T, b1, w2T, b2):
    hh = jnp.maximum(_dot(h, w1T) + b1, 0.0)
    return _dot(hh, w2T) + b2


_SEG_KEYS = ("w1cat", "b1cat6", "w2bd", "b2cat", "wg", "bg")
_TC_ARG_KEYS = (
    ["node", "wproj2", "pab_b", "pbc_b", "wrelp"]
    + ["ab_" + k for k in _SEG_KEYS] + ["bc_" + k for k in _SEG_KEYS]
    + ["hab_w1T", "hab_b1", "hab_w2T", "hab_b2",
       "hbc_w1T", "hbc_b1", "hbc_w2T", "hbc_b2"])


def _tc_body(*refs):
    w = {k: r[...] for k, r in zip(_TC_ARG_KEYS, refs[:len(_TC_ARG_KEYS)])}
    ab_out, bc_out = refs[len(_TC_ARG_KEYS):]
    S = _S
    node = w["node"].reshape(4 * S, EMB)
    ab = {k[3:]: w[k] for k in w if k.startswith("ab_")}
    bc = {k[3:]: w[k] for k in w if k.startswith("bc_")}

    XP = _dot(node, w["wproj2"])
    x_ab = jnp.maximum(XP[:, 0:HID] + w["pab_b"], 0.0)
    xbcn = XP[:, HID:2 * HID]

    s_ab = _gnn(x_ab, ab)
    h_ab = s_ab[0:S] - s_ab[2 * S:3 * S]
    logits_ab = _head(h_ab, w["hab_w1T"], w["hab_b1"], w["hab_w2T"],
                      w["hab_b2"])
    ab_out[...] = logits_ab

    mx = jnp.max(logits_ab, axis=1, keepdims=True)
    iota = lax.broadcasted_iota(jnp.int32, (S, NREL), 1)
    cand = jnp.where(logits_ab >= mx, iota, NREL)
    rel = jnp.min(cand, axis=1, keepdims=True)
    oh = (iota == rel).astype(jnp.float32)
    t = _dot(oh, w["wrelp"])

    r_rep = jnp.concatenate([t, t, t, t], axis=0)
    x_bc = jnp.maximum(xbcn + r_rep + w["pbc_b"], 0.0)
    s_bc = _gnn(x_bc, bc)
    h_bc = s_bc[2 * S:3 * S] - s_bc[3 * S:4 * S]
    bc_out[...] = _head(h_bc, w["hbc_w1T"], w["hbc_b1"], w["hbc_w2T"],
                        w["hbc_b2"])


def _tc_forward(args, interpret=False):
    def spec(k):
        a = args[k]
        if k == "node":
            return pl.BlockSpec((4, _S, EMB), lambda i: (0, i, 0))
        nd = a.ndim
        return pl.BlockSpec(a.shape, lambda i, _n=nd: (0,) * _n)

    return pl.pallas_call(
        _tc_body,
        grid=(_GRID,),
        in_specs=[spec(k) for k in _TC_ARG_KEYS],
        out_specs=[pl.BlockSpec((_S, NREL), lambda i: (i, 0)),
                   pl.BlockSpec((_S, NREL), lambda i: (i, 0))],
        out_shape=[jax.ShapeDtypeStruct((B, NREL), jnp.float32),
                   jax.ShapeDtypeStruct((B, NREL), jnp.float32)],
        interpret=interpret,
    )(*[args[k] for k in _TC_ARG_KEYS])


def _pack_seg(msg_W1, msg_b1, msg_W2, msg_b2, att_W1, att_b1, att_W2, att_b2,
              gru_Wih, gru_Whh, gru_bih, gru_bhh):
    w = {}
    AH = att_W1.shape[0]
    w["w1cat"] = jnp.concatenate([msg_W1[:, :EMB].T,
                                  att_W1[:, :EMB].T], axis=1)
    b1m = msg_b1[None, :] + msg_W1[:, EMB:EMB + 6].T
    b1a = att_b1[None, :] + att_W1[:, EMB:EMB + 6].T
    w["b1cat6"] = jnp.repeat(jnp.concatenate([b1m, b1a], axis=1), _S, axis=0)
    z_mu = jnp.zeros((HID, HID), jnp.float32)
    z_al = jnp.zeros((AH, HID), jnp.float32)
    w["w2bd"] = jnp.block([[msg_W2.T, z_mu], [z_al, att_W2.T]])
    w["b2cat"] = jnp.concatenate([msg_b2, att_b2])[None, :]
    wih = gru_Wih.T
    whh = gru_Whh.T
    zh = jnp.zeros((HID, HID), jnp.float32)
    w["wg"] = jnp.block([
        [wih[:, 0:HID], wih[:, HID:2 * HID], wih[:, 2 * HID:], zh],
        [whh[:, 0:HID], whh[:, HID:2 * HID], zh, whh[:, 2 * HID:]],
    ])
    w["bg"] = jnp.concatenate([
        gru_bih[0:HID] + gru_bhh[0:HID],
        gru_bih[HID:2 * HID] + gru_bhh[HID:2 * HID],
        gru_bih[2 * HID:], gru_bhh[2 * HID:]])[None, :]
    return w


def _assemble_tc_args(node4, rel_emb, p):
    args = {"node": node4}
    args["wproj2"] = jnp.concatenate([p["proj_ab_W"].T,
                                      p["proj_bc_W"][:, :EMB].T], axis=1)
    args["pab_b"] = p["proj_ab_b"][None, :]
    args["pbc_b"] = p["proj_bc_b"][None, :]
    args["wrelp"] = rel_emb @ p["proj_bc_W"][:, EMB:].T
    for pre in ("ab", "bc"):
        seg = _pack_seg(*[p[f"{pre}_{n}"] for n in (
            "msg_W1", "msg_b1", "msg_W2", "msg_b2",
            "att_W1", "att_b1", "att_W2", "att_b2",
            "gru_Wih", "gru_Whh", "gru_bih", "gru_bhh")])
        for k, v in seg.items():
            args[f"{pre}_{k}"] = v
    for pre, tag in (("head_ab", "hab"), ("head_bc", "hbc")):
        args[f"{tag}_w1T"] = p[f"{pre}_W1"].T
        args[f"{tag}_b1"] = p[f"{pre}_b1"][None, :]
        args[f"{tag}_w2T"] = p[f"{pre}_W2"].T
        args[f"{tag}_b2"] = p[f"{pre}_b2"][None, :]
    return args


def kernel(a_ids, event_ids, b_ids, c_ids, ent_emb, rel_emb,
           proj_ab_W, proj_ab_b, proj_bc_W, proj_bc_b,
           ab_msg_W1, ab_msg_b1, ab_msg_W2, ab_msg_b2,
           ab_att_W1, ab_att_b1, ab_att_W2, ab_att_b2,
           ab_gru_Wih, ab_gru_Whh, ab_gru_bih, ab_gru_bhh,
           bc_msg_W1, bc_msg_b1, bc_msg_W2, bc_msg_b2,
           bc_att_W1, bc_att_b1, bc_att_W2, bc_att_b2,
           bc_gru_Wih, bc_gru_Whh, bc_gru_bih, bc_gru_bhh,
           head_ab_W1, head_ab_b1, head_ab_W2, head_ab_b2,
           head_bc_W1, head_bc_b1, head_bc_W2, head_bc_b2):
    p = dict(locals())
    ids_all = jnp.concatenate(
        [a_ids, event_ids, b_ids, c_ids]).astype(jnp.int32)
    gathered = _make_sc_gather()(ids_all, ent_emb)
    node4 = gathered.reshape(4, B, EMB)
    args = _assemble_tc_args(node4, rel_emb, p)
    logits_ab, logits_bc = _tc_forward(args)
    return logits_ab, logits_bc
